# baseline (speedup 1.0000x reference)
import functools

import jax
import jax.numpy as jnp
from jax import lax
from jax.experimental import pallas as pl
from jax.experimental.pallas import tpu as pltpu
from jax.experimental.pallas import tpu_sc as plsc

VOCAB = 30522
EMBED = 128
HIDDEN = 64
PACKED = HIDDEN // 2
B = 1024
S = 512
ROWS = 3 * B
CHUNK = 128
NCHUNK = S // CHUNK
NBUF = 8

VOCABP = 30528
QROWS = VOCABP // 4
BLKR = 1272


def _round_to_bf16_bits(x):
    u = lax.bitcast_convert_type(x, jnp.int32)
    r = u + jnp.int32(0x7FFF) + ((u >> 16) & jnp.int32(1))
    return r & jnp.int32(-65536)


def _proj_body(t0, t1, t2, t3, we_ref, wo_ref, out_ref):
    we = we_ref[...].astype(jnp.bfloat16)
    wo = wo_ref[...].astype(jnp.bfloat16)
    parts = []
    for tr in (t0, t1, t2, t3):
        tab = tr[...].astype(jnp.bfloat16)
        pe = jnp.dot(tab, we, preferred_element_type=jnp.float32)
        po = jnp.dot(tab, wo, preferred_element_type=jnp.float32)
        parts.append(_round_to_bf16_bits(po)
                     | lax.shift_right_logical(_round_to_bf16_bits(pe), 16))
    out_ref[...] = jnp.concatenate(parts, axis=1)


def _project_table(table, we, wo):
    nblk = QROWS // BLKR
    in_specs = [
        pl.BlockSpec((BLKR, EMBED), (lambda i, j=j: (j * nblk + i, 0)))
        for j in range(4)
    ] + [
        pl.BlockSpec((EMBED, PACKED), lambda i: (0, 0)),
        pl.BlockSpec((EMBED, PACKED), lambda i: (0, 0)),
    ]
    return pl.pallas_call(
        _proj_body,
        grid=(nblk,),
        in_specs=in_specs,
        out_specs=pl.BlockSpec((BLKR, 4 * PACKED), lambda i: (i, 0)),
        out_shape=jax.ShapeDtypeStruct((QROWS, 4 * PACKED), jnp.int32),
    )(table, table, table, table, we, wo)


def _make_sc_pool():
    info = plsc.get_sparse_core_info()
    nc, ns = info.num_cores, info.num_subcores
    nw = nc * ns
    rpb = B // nw
    rpw = 3 * rpb

    mesh = plsc.VectorSubcoreMesh(core_axis_name="c", subcore_axis_name="s")

    @functools.partial(
        pl.kernel,
        mesh=mesh,
        out_type=jax.ShapeDtypeStruct((ROWS, HIDDEN), jnp.float32),
        scratch_types=[
            pltpu.VMEM((rpw, NCHUNK, CHUNK), jnp.int32),
            *[pltpu.VMEM((CHUNK, PACKED), jnp.int32) for _ in range(NBUF)],
            pltpu.VMEM((rpw, HIDDEN), jnp.float32),
            pltpu.VMEM((HIDDEN,), jnp.float32),
            *[pltpu.SemaphoreType.DMA for _ in range(NBUF)],
        ],
        compiler_params=pltpu.CompilerParams(
            use_tc_tiling_on_sc=False, needs_layout_passes=False),
    )
    def sc_pool(a_hbm, p_hbm, n_hbm, proj_hbm, bias_hbm, out_hbm,
                idx_v, *rest):
        wid = lax.axis_index("s") * nc + lax.axis_index("c")
        gbufs = rest[:NBUF]
        out_v, bias_v = rest[NBUF], rest[NBUF + 1]
        sems = rest[NBUF + 2:]
        pltpu.sync_copy(bias_hbm, bias_v)
        for t, ids_hbm in enumerate((a_hbm, p_hbm, n_hbm)):
            pltpu.async_copy(ids_hbm.at[pl.ds(wid * rpb, rpb)],
                             idx_v.at[pl.ds(t * rpb, rpb)], sems[t])
        for t in range(3):
            pltpu.make_async_copy(
                a_hbm.at[pl.ds(0, rpb)],
                idx_v.at[pl.ds(t * rpb, rpb)], sems[t]).wait()
        def _xform_chunk(idx_ref, r, c):
            @plsc.parallel_loop(0, CHUNK, step=16)
            def _(i):
                v = idx_ref[r, c, pl.ds(i, 16)]
                q = v // jnp.int32(QROWS)
                idx_ref[r, c, pl.ds(i, 16)] = (
                    (v << 2) - q * jnp.int32(4 * QROWS - 1))

        for k in range(NBUF - 1):
            _xform_chunk(idx_v, k // NCHUNK, k % NCHUNK)
            pltpu.async_copy(proj_hbm.at[idx_v.at[k // NCHUNK, k % NCHUNK]],
                             gbufs[k], sems[k])
        dummy = proj_hbm.at[pl.ds(0, CHUNK)]

        grows = NBUF // NCHUNK

        def group_body(g, _):
            base_r = g * grows
            for half in range(grows):
                r = base_r + half
                accs = tuple(jnp.zeros((16,), jnp.float32) for _ in range(4))
                for c in range(NCHUNK):
                    m = half * NCHUNK + c
                    gcur, scur = gbufs[m], sems[m]
                    pltpu.make_async_copy(dummy, gcur, scur).wait()
                    tr_off = (m + NBUF - 1) // NCHUNK
                    tc = (m + NBUF - 1) % NCHUNK
                    tb = (m + NBUF - 1) % NBUF
                    tr = base_r + tr_off

                    @pl.when(tr < rpw)
                    def _():
                        _xform_chunk(idx_v, tr, tc)
                        pltpu.async_copy(
                            proj_hbm.at[idx_v.at[tr, tc]],
                            gbufs[tb], sems[tb])

                    @plsc.parallel_loop(0, CHUNK, unroll=16, carry=accs)
                    def accs(i, carry):
                        a0, a1, a2, a3 = carry
                        w0 = gcur[i, pl.ds(0, 16)]
                        w1 = gcur[i, pl.ds(16, 16)]
                        a0 += plsc.bitcast(w0 << 16, jnp.float32)
                        a1 += plsc.bitcast(w0 & jnp.int32(-65536), jnp.float32)
                        a2 += plsc.bitcast(w1 << 16, jnp.float32)
                        a3 += plsc.bitcast(w1 & jnp.int32(-65536), jnp.float32)
                        return a0, a1, a2, a3

                inv = jnp.float32(1.0 / S)
                for q in range(4):
                    val = jnp.maximum(
                        accs[q] * inv + bias_v[pl.ds(q * 16, 16)], 0.0)
                    out_v[r, pl.ds(q * 16, 16)] = val
            return 0

        lax.fori_loop(0, rpw // (NBUF // NCHUNK), group_body, 0)
        for t in range(3):
            pltpu.sync_copy(out_v.at[pl.ds(t * rpb, rpb)],
                            out_hbm.at[pl.ds(t * B + wid * rpb, rpb)])

    return sc_pool


def kernel(anchor_input_ids, anchor_attention_mask,
           positive_input_ids, positive_attention_mask,
           negative_input_ids, negative_attention_mask,
           emb_table, fc_W, fc_b):
    def prep(ids):
        return ids.astype(jnp.int32).reshape(B, NCHUNK, CHUNK)

    we = jnp.concatenate([fc_W[:, 0:16], fc_W[:, 32:48]], axis=1)
    wo = jnp.concatenate([fc_W[:, 16:32], fc_W[:, 48:64]], axis=1)
    proj = _project_table(emb_table, we, wo).reshape(VOCABP, PACKED)
    return _make_sc_pool()(
        prep(anchor_input_ids), prep(positive_input_ids),
        prep(negative_input_ids), proj, fc_b)


# speedup vs baseline: 2.1014x; 2.1014x over previous
import functools

import jax
import jax.numpy as jnp
from jax import lax
from jax.experimental import pallas as pl
from jax.experimental.pallas import tpu as pltpu
from jax.experimental.pallas import tpu_sc as plsc

VOCAB = 30522
EMBED = 128
HIDDEN = 64
PACKED = HIDDEN // 2
B = 1024
S = 512
ROWS = 3 * B
CHUNK = 128
NCHUNK = S // CHUNK
NBUF = 8

VOCABP = 30528
QROWS = VOCABP // 4
BLKR = 1272


def _round_to_bf16_bits(x):
    u = lax.bitcast_convert_type(x, jnp.int32)
    r = u + jnp.int32(0x7FFF) + ((u >> 16) & jnp.int32(1))
    return r & jnp.int32(-65536)


def _proj_body(t0, t1, t2, t3, we_ref, wo_ref, out_ref):
    we = we_ref[...].astype(jnp.bfloat16)
    wo = wo_ref[...].astype(jnp.bfloat16)
    parts = []
    for tr in (t0, t1, t2, t3):
        tab = tr[...].astype(jnp.bfloat16)
        pe = jnp.dot(tab, we, preferred_element_type=jnp.float32)
        po = jnp.dot(tab, wo, preferred_element_type=jnp.float32)
        parts.append(_round_to_bf16_bits(po)
                     | lax.shift_right_logical(_round_to_bf16_bits(pe), 16))
    out_ref[...] = jnp.concatenate(parts, axis=1)


def _project_table(table, we, wo):
    nblk = QROWS // BLKR
    in_specs = [
        pl.BlockSpec((BLKR, EMBED), (lambda i, j=j: (j * nblk + i, 0)))
        for j in range(4)
    ] + [
        pl.BlockSpec((EMBED, PACKED), lambda i: (0, 0)),
        pl.BlockSpec((EMBED, PACKED), lambda i: (0, 0)),
    ]
    return pl.pallas_call(
        _proj_body,
        grid=(nblk,),
        in_specs=in_specs,
        out_specs=pl.BlockSpec((BLKR, 4 * PACKED), lambda i: (i, 0)),
        out_shape=jax.ShapeDtypeStruct((QROWS, 4 * PACKED), jnp.int32),
    )(table, table, table, table, we, wo)


def _make_sc_pool():
    info = plsc.get_sparse_core_info()
    nc, ns = info.num_cores, info.num_subcores
    nw = nc * ns
    rpb = B // nw
    rpw = 3 * rpb

    mesh = plsc.VectorSubcoreMesh(core_axis_name="c", subcore_axis_name="s")

    @functools.partial(
        pl.kernel,
        mesh=mesh,
        out_type=jax.ShapeDtypeStruct((ROWS, HIDDEN), jnp.float32),
        scratch_types=[
            pltpu.VMEM((rpw, NCHUNK, CHUNK), jnp.int32),
            *[pltpu.VMEM((CHUNK, PACKED), jnp.int32) for _ in range(NBUF)],
            pltpu.VMEM((rpw, HIDDEN), jnp.float32),
            pltpu.VMEM((HIDDEN,), jnp.float32),
            *[pltpu.SemaphoreType.DMA for _ in range(NBUF)],
        ],
        compiler_params=pltpu.CompilerParams(
            use_tc_tiling_on_sc=False, needs_layout_passes=False),
    )
    def sc_pool(a_hbm, p_hbm, n_hbm, proj_hbm, bias_hbm, out_hbm,
                idx_v, *rest):
        wid = lax.axis_index("s") * nc + lax.axis_index("c")
        gbufs = rest[:NBUF]
        out_v, bias_v = rest[NBUF], rest[NBUF + 1]
        sems = rest[NBUF + 2:]
        pltpu.sync_copy(bias_hbm, bias_v)
        for t, ids_hbm in enumerate((a_hbm, p_hbm, n_hbm)):
            pltpu.async_copy(ids_hbm.at[pl.ds(wid * rpb, rpb)],
                             idx_v.at[pl.ds(t * rpb, rpb)], sems[t])
        for t in range(3):
            pltpu.make_async_copy(
                a_hbm.at[pl.ds(0, rpb)],
                idx_v.at[pl.ds(t * rpb, rpb)], sems[t]).wait()
        d = jnp.int32(4 * QROWS - 1)
        z = jnp.zeros((16,), jnp.int32)

        def _xform_chunk(idx_ref, r, c):
            @plsc.parallel_loop(0, CHUNK, step=16, unroll=4)
            def _(i):
                v = idx_ref[r, c, pl.ds(i, 16)]
                k = v << 2
                k -= jnp.where(v >= QROWS, d, z)
                k -= jnp.where(v >= 2 * QROWS, d, z)
                k -= jnp.where(v >= 3 * QROWS, d, z)
                idx_ref[r, c, pl.ds(i, 16)] = k

        for k in range(NBUF - 1):
            _xform_chunk(idx_v, k // NCHUNK, k % NCHUNK)
            pltpu.async_copy(proj_hbm.at[idx_v.at[k // NCHUNK, k % NCHUNK]],
                             gbufs[k], sems[k])
        dummy = proj_hbm.at[pl.ds(0, CHUNK)]

        grows = NBUF // NCHUNK

        def group_body(g, _):
            base_r = g * grows
            for half in range(grows):
                r = base_r + half
                accs = tuple(jnp.zeros((16,), jnp.float32) for _ in range(4))
                for c in range(NCHUNK):
                    m = half * NCHUNK + c
                    gcur, scur = gbufs[m], sems[m]
                    pltpu.make_async_copy(dummy, gcur, scur).wait()
                    tr_off = (m + NBUF - 1) // NCHUNK
                    tc = (m + NBUF - 1) % NCHUNK
                    tb = (m + NBUF - 1) % NBUF
                    tr = base_r + tr_off

                    @pl.when(tr < rpw)
                    def _():
                        _xform_chunk(idx_v, tr, tc)
                        pltpu.async_copy(
                            proj_hbm.at[idx_v.at[tr, tc]],
                            gbufs[tb], sems[tb])

                    @plsc.parallel_loop(0, CHUNK, unroll=16, carry=accs)
                    def accs(i, carry):
                        a0, a1, a2, a3 = carry
                        w0 = gcur[i, pl.ds(0, 16)]
                        w1 = gcur[i, pl.ds(16, 16)]
                        a0 += plsc.bitcast(w0 << 16, jnp.float32)
                        a1 += plsc.bitcast(w0 & jnp.int32(-65536), jnp.float32)
                        a2 += plsc.bitcast(w1 << 16, jnp.float32)
                        a3 += plsc.bitcast(w1 & jnp.int32(-65536), jnp.float32)
                        return a0, a1, a2, a3

                inv = jnp.float32(1.0 / S)
                for q in range(4):
                    val = jnp.maximum(
                        accs[q] * inv + bias_v[pl.ds(q * 16, 16)], 0.0)
                    out_v[r, pl.ds(q * 16, 16)] = val
            return 0

        lax.fori_loop(0, rpw // (NBUF // NCHUNK), group_body, 0)
        for t in range(3):
            pltpu.sync_copy(out_v.at[pl.ds(t * rpb, rpb)],
                            out_hbm.at[pl.ds(t * B + wid * rpb, rpb)])

    return sc_pool


def kernel(anchor_input_ids, anchor_attention_mask,
           positive_input_ids, positive_attention_mask,
           negative_input_ids, negative_attention_mask,
           emb_table, fc_W, fc_b):
    def prep(ids):
        return ids.astype(jnp.int32).reshape(B, NCHUNK, CHUNK)

    we = jnp.concatenate([fc_W[:, 0:16], fc_W[:, 32:48]], axis=1)
    wo = jnp.concatenate([fc_W[:, 16:32], fc_W[:, 48:64]], axis=1)
    proj = _project_table(emb_table, we, wo).reshape(VOCABP, PACKED)
    return _make_sc_pool()(
        prep(anchor_input_ids), prep(positive_input_ids),
        prep(negative_input_ids), proj, fc_b)


# speedup vs baseline: 2.1934x; 1.0438x over previous
import functools

import jax
import jax.numpy as jnp
from jax import lax
from jax.experimental import pallas as pl
from jax.experimental.pallas import tpu as pltpu
from jax.experimental.pallas import tpu_sc as plsc

VOCAB = 30522
EMBED = 128
HIDDEN = 64
PACKED = HIDDEN // 2
B = 1024
S = 512
ROWS = 3 * B
CHUNK = 128
NCHUNK = S // CHUNK
NBUF = 8

VOCABP = 30528
QROWS = VOCABP // 4
BLKR = 1272


def _round_to_bf16_bits(x):
    u = lax.bitcast_convert_type(x, jnp.int32)
    r = u + jnp.int32(0x7FFF) + ((u >> 16) & jnp.int32(1))
    return r & jnp.int32(-65536)


def _proj_body(t0, t1, t2, t3, we_ref, wo_ref, out_ref):
    we = we_ref[...].astype(jnp.bfloat16)
    wo = wo_ref[...].astype(jnp.bfloat16)
    parts = []
    for tr in (t0, t1, t2, t3):
        tab = tr[...].astype(jnp.bfloat16)
        pe = jnp.dot(tab, we, preferred_element_type=jnp.float32)
        po = jnp.dot(tab, wo, preferred_element_type=jnp.float32)
        parts.append(_round_to_bf16_bits(po)
                     | lax.shift_right_logical(_round_to_bf16_bits(pe), 16))
    out_ref[...] = jnp.concatenate(parts, axis=1)


def _project_table(table, we, wo):
    nblk = QROWS // BLKR
    in_specs = [
        pl.BlockSpec((BLKR, EMBED), (lambda i, j=j: (j * nblk + i, 0)))
        for j in range(4)
    ] + [
        pl.BlockSpec((EMBED, PACKED), lambda i: (0, 0)),
        pl.BlockSpec((EMBED, PACKED), lambda i: (0, 0)),
    ]
    return pl.pallas_call(
        _proj_body,
        grid=(nblk,),
        in_specs=in_specs,
        out_specs=pl.BlockSpec((BLKR, 4 * PACKED), lambda i: (i, 0)),
        out_shape=jax.ShapeDtypeStruct((QROWS, 4 * PACKED), jnp.int32),
    )(table, table, table, table, we, wo)


def _make_sc_pool():
    info = plsc.get_sparse_core_info()
    nc, ns = info.num_cores, info.num_subcores
    nw = nc * ns
    rpb = B // nw
    rpw = 3 * rpb

    mesh = plsc.VectorSubcoreMesh(core_axis_name="c", subcore_axis_name="s")

    @functools.partial(
        pl.kernel,
        mesh=mesh,
        out_type=jax.ShapeDtypeStruct((ROWS, HIDDEN), jnp.float32),
        scratch_types=[
            pltpu.VMEM((rpw, NCHUNK, CHUNK), jnp.int32),
            *[pltpu.VMEM((CHUNK, PACKED), jnp.int32) for _ in range(NBUF)],
            pltpu.VMEM((rpw, HIDDEN), jnp.float32),
            pltpu.VMEM((HIDDEN,), jnp.float32),
            *[pltpu.SemaphoreType.DMA for _ in range(NBUF)],
        ],
        compiler_params=pltpu.CompilerParams(
            use_tc_tiling_on_sc=False, needs_layout_passes=False),
    )
    def sc_pool(a_hbm, p_hbm, n_hbm, proj_hbm, bias_hbm, out_hbm,
                idx_v, *rest):
        wid = lax.axis_index("s") * nc + lax.axis_index("c")
        gbufs = rest[:NBUF]
        out_v, bias_v = rest[NBUF], rest[NBUF + 1]
        sems = rest[NBUF + 2:]
        pltpu.sync_copy(bias_hbm, bias_v)
        for t, ids_hbm in enumerate((a_hbm, p_hbm, n_hbm)):
            pltpu.async_copy(ids_hbm.at[pl.ds(wid * rpb, rpb)],
                             idx_v.at[pl.ds(t * rpb, rpb)], sems[t])
        for t in range(3):
            pltpu.make_async_copy(
                a_hbm.at[pl.ds(0, rpb)],
                idx_v.at[pl.ds(t * rpb, rpb)], sems[t]).wait()
        for k in range(NBUF - 1):
            pltpu.async_copy(proj_hbm.at[idx_v.at[k // NCHUNK, k % NCHUNK]],
                             gbufs[k], sems[k])
        dummy = proj_hbm.at[pl.ds(0, CHUNK)]

        grows = NBUF // NCHUNK

        def group_body(g, _):
            base_r = g * grows
            for half in range(grows):
                r = base_r + half
                accs = tuple(jnp.zeros((16,), jnp.float32) for _ in range(4))
                for c in range(NCHUNK):
                    m = half * NCHUNK + c
                    gcur, scur = gbufs[m], sems[m]
                    pltpu.make_async_copy(dummy, gcur, scur).wait()
                    tr_off = (m + NBUF - 1) // NCHUNK
                    tc = (m + NBUF - 1) % NCHUNK
                    tb = (m + NBUF - 1) % NBUF
                    tr = base_r + tr_off

                    @pl.when(tr < rpw)
                    def _():
                        pltpu.async_copy(
                            proj_hbm.at[idx_v.at[tr, tc]],
                            gbufs[tb], sems[tb])

                    @plsc.parallel_loop(0, CHUNK, unroll=16, carry=accs)
                    def accs(i, carry):
                        a0, a1, a2, a3 = carry
                        w0 = gcur[i, pl.ds(0, 16)]
                        w1 = gcur[i, pl.ds(16, 16)]
                        a0 += plsc.bitcast(w0 << 16, jnp.float32)
                        a1 += plsc.bitcast(w0 & jnp.int32(-65536), jnp.float32)
                        a2 += plsc.bitcast(w1 << 16, jnp.float32)
                        a3 += plsc.bitcast(w1 & jnp.int32(-65536), jnp.float32)
                        return a0, a1, a2, a3

                inv = jnp.float32(1.0 / S)
                for q in range(4):
                    val = jnp.maximum(
                        accs[q] * inv + bias_v[pl.ds(q * 16, 16)], 0.0)
                    out_v[r, pl.ds(q * 16, 16)] = val
            return 0

        lax.fori_loop(0, rpw // (NBUF // NCHUNK), group_body, 0)
        for t in range(3):
            pltpu.sync_copy(out_v.at[pl.ds(t * rpb, rpb)],
                            out_hbm.at[pl.ds(t * B + wid * rpb, rpb)])

    return sc_pool


def kernel(anchor_input_ids, anchor_attention_mask,
           positive_input_ids, positive_attention_mask,
           negative_input_ids, negative_attention_mask,
           emb_table, fc_W, fc_b):
    def prep(ids):
        v = ids.astype(jnp.int32)
        k = (v << 2) - (v // QROWS) * (4 * QROWS - 1)
        return k.reshape(B, NCHUNK, CHUNK)

    we = jnp.concatenate([fc_W[:, 0:16], fc_W[:, 32:48]], axis=1)
    wo = jnp.concatenate([fc_W[:, 16:32], fc_W[:, 48:64]], axis=1)
    proj = _project_table(emb_table, we, wo).reshape(VOCABP, PACKED)
    return _make_sc_pool()(
        prep(anchor_input_ids), prep(positive_input_ids),
        prep(negative_input_ids), proj, fc_b)
